# baseline (speedup 1.0000x reference)
import functools

import jax
import jax.numpy as jnp
from jax import lax
from jax.experimental import pallas as pl
from jax.experimental.pallas import tpu as pltpu
from jax.experimental.pallas import tpu_sc as plsc

N = 10000
D = 128
E = 320000
NC = 2
NS = 16
NW = NC * NS
L = 16

N_PAD = 10240
ROWS_PER_TILE = N_PAD // NS
CHUNK = 128
E_TILE = 10240
CHUNKS = E_TILE // CHUNK
E_PAD = NW * E_TILE
TOTAL_CHUNKS = E_PAD // CHUNK
PAD_IDX = N
NBUF = 2
C0_CHUNKS = 112
C1_CHUNKS = 48

_mesh = plsc.VectorSubcoreMesh(core_axis_name="c", subcore_axis_name="s")
_sc_params = pltpu.CompilerParams(needs_layout_passes=False)


@functools.partial(
    pl.kernel,
    out_type=jax.ShapeDtypeStruct((NW, N_PAD), jnp.float32),
    mesh=_mesh,
    scratch_types=[
        pltpu.VMEM((CHUNKS, CHUNK), jnp.int32),
        pltpu.VMEM((N_PAD,), jnp.float32),
    ],
    compiler_params=_sc_params,
)
def _deg_kernel(dst_hbm, out_hbm, dst_v, deg_v):
    c = lax.axis_index("c")
    s = lax.axis_index("s")
    wid = s * NC + c

    zeros16 = jnp.zeros((L,), jnp.float32)

    def zero_body(i, _):
        deg_v[pl.ds(i * L, L)] = zeros16
        return 0

    lax.fori_loop(0, N_PAD // L, zero_body, 0)

    pltpu.sync_copy(dst_hbm.at[wid], dst_v)

    ones16 = jnp.ones((L,), jnp.float32)

    def chunk_body(j, _):
        for k in range(CHUNK // L):
            idx = dst_v[j, pl.ds(k * L, L)]
            plsc.addupdate_scatter(deg_v, [idx], ones16)
        return 0

    lax.fori_loop(0, CHUNKS, chunk_body, 0)

    pltpu.sync_copy(deg_v, out_hbm.at[wid])


@functools.partial(
    pl.kernel,
    out_type=jax.ShapeDtypeStruct((NW, ROWS_PER_TILE, D), jnp.float32),
    mesh=_mesh,
    scratch_types=[
        [pltpu.VMEM((CHUNK,), jnp.int32)] * NBUF,
        [pltpu.VMEM((CHUNK,), jnp.int32)] * NBUF,
        [pltpu.VMEM((CHUNK, D), jnp.float32)] * NBUF,
        pltpu.VMEM_SHARED((N_PAD, D), jnp.float32),
        [pltpu.SemaphoreType.DMA] * NBUF,
        [pltpu.SemaphoreType.DMA] * NBUF,
        [pltpu.SemaphoreType.DMA] * NBUF,
        [pltpu.SemaphoreType.DMA] * NBUF,
    ],
    compiler_params=_sc_params,
)
def _agg_kernel(g_hbm, src_hbm, dst_hbm, out_hbm, sidx, didx, bufs,
                acc, isems, dsems, gsems, ssems):
    c = lax.axis_index("c")
    s = lax.axis_index("s")
    owid = c * NS + s
    n = jnp.where(c == 0, C0_CHUNKS, C1_CHUNKS)
    base = jnp.where(c == 0, s * C0_CHUNKS,
                     NS * C0_CHUNKS + s * C1_CHUNKS)

    zeros16 = jnp.zeros((L,), jnp.float32)

    def zbuf_body(i, _):
        for k in range(D // L):
            bufs[0][i, pl.ds(k * L, L)] = zeros16
        return 0

    lax.fori_loop(0, CHUNK, zbuf_body, 0)

    def zacc_body(t, _):
        pltpu.sync_copy(
            bufs[0], acc.at[pl.ds(s * ROWS_PER_TILE + t * CHUNK, CHUNK)])
        return 0

    lax.fori_loop(0, ROWS_PER_TILE // CHUNK, zacc_body, 0)

    plsc.subcore_barrier()

    for b in range(NBUF):
        pltpu.async_copy(src_hbm.at[base + b], sidx[b], isems[b])
        pltpu.async_copy(dst_hbm.at[base + b], didx[b], dsems[b])
    for b in range(NBUF):
        pltpu.make_async_copy(src_hbm.at[base + b], sidx[b],
                              isems[b]).wait()
        pltpu.async_copy(g_hbm.at[sidx[b]], bufs[b], gsems[b])

    def step(jj, _):
        for b in range(NBUF):
            k = jj * NBUF + b
            j = base + k
            pltpu.make_async_copy(g_hbm.at[sidx[b]], bufs[b],
                                  gsems[b]).wait()
            pltpu.make_async_copy(dst_hbm.at[j], didx[b], dsems[b]).wait()
            pltpu.async_copy(bufs[b], acc.at[didx[b]], ssems[b], add=True)
            nk = k + NBUF

            @pl.when(nk < n)
            def _():
                pltpu.async_copy(src_hbm.at[j + NBUF], sidx[b], isems[b])
                pltpu.make_async_copy(bufs[b], acc.at[didx[b]],
                                      ssems[b]).wait()
                pltpu.async_copy(dst_hbm.at[j + NBUF], didx[b], dsems[b])
                pltpu.make_async_copy(src_hbm.at[j + NBUF], sidx[b],
                                      isems[b]).wait()
                pltpu.async_copy(g_hbm.at[sidx[b]], bufs[b], gsems[b])

        return 0

    lax.fori_loop(0, n // NBUF, step, 0)

    for b in range(NBUF):
        pltpu.make_async_copy(bufs[b], acc.at[didx[b]], ssems[b]).wait()

    plsc.subcore_barrier()

    pltpu.sync_copy(acc.at[pl.ds(s * ROWS_PER_TILE, ROWS_PER_TILE)],
                    out_hbm.at[owid])


def _deg_combine_body(degp_ref, ds_ref, di_ref):
    deg = jnp.sum(degp_ref[...], axis=0, keepdims=True) + 1.0
    ds_ref[...] = lax.rsqrt(deg)
    di_ref[...] = 1.0 / deg


_deg_combine = pl.pallas_call(
    _deg_combine_body,
    out_shape=[jax.ShapeDtypeStruct((1, N_PAD), jnp.float32)] * 2,
)

BLK = 512
GRID = N_PAD // BLK

_row_spec = pl.BlockSpec((BLK, D), lambda i: (i, 0))
_col_spec = pl.BlockSpec((BLK, 1), lambda i: (i, 0))
_w_spec = pl.BlockSpec((D, D), lambda i: (0, 0))
_b_spec = pl.BlockSpec((1, D), lambda i: (0, 0))


def _mm1_body(x_ref, w_ref, ds_ref, h_ref, g_ref):
    h = jnp.dot(x_ref[...], w_ref[...], preferred_element_type=jnp.float32,
                precision=lax.Precision.HIGHEST)
    h_ref[...] = h
    g_ref[...] = h * ds_ref[...]


_mm1 = pl.pallas_call(
    _mm1_body,
    grid=(GRID,),
    in_specs=[_row_spec, _w_spec, _col_spec],
    out_specs=[_row_spec, _row_spec],
    out_shape=[jax.ShapeDtypeStruct((N_PAD, D), jnp.float32)] * 2,
)


def _comb_body(a0_ref, a1_ref, h_ref, ds_ref, di_ref, b_ref, w_ref,
               hn_ref, gn_ref):
    x = (a0_ref[...] + a1_ref[...]) * ds_ref[...] \
        + h_ref[...] * di_ref[...] + b_ref[...]
    x = jnp.maximum(x, 0.0)
    hn = jnp.dot(x, w_ref[...], preferred_element_type=jnp.float32,
                 precision=lax.Precision.HIGHEST)
    hn_ref[...] = hn
    gn_ref[...] = hn * ds_ref[...]


_comb = pl.pallas_call(
    _comb_body,
    grid=(GRID,),
    in_specs=[_row_spec, _row_spec, _row_spec, _col_spec, _col_spec,
              _b_spec, _w_spec],
    out_specs=[_row_spec, _row_spec],
    out_shape=[jax.ShapeDtypeStruct((N_PAD, D), jnp.float32)] * 2,
)


def _final_body(a0_ref, a1_ref, h_ref, ds_ref, di_ref, b_ref, o_ref):
    o_ref[...] = (a0_ref[...] + a1_ref[...]) * ds_ref[...] \
        + h_ref[...] * di_ref[...] + b_ref[...]


_final = pl.pallas_call(
    _final_body,
    grid=(GRID,),
    in_specs=[_row_spec, _row_spec, _row_spec, _col_spec, _col_spec, _b_spec],
    out_specs=_row_spec,
    out_shape=jax.ShapeDtypeStruct((N_PAD, D), jnp.float32),
)


def kernel(x, edge_index, W1, b1, W2, b2, W3, b3):
    src = edge_index[0].astype(jnp.int32)
    dst = edge_index[1].astype(jnp.int32)
    pad = jnp.full((E_PAD - E,), PAD_IDX, jnp.int32)
    srcp = jnp.concatenate([src, pad]).reshape(TOTAL_CHUNKS, CHUNK)
    dstp = jnp.concatenate([dst, pad]).reshape(TOTAL_CHUNKS, CHUNK)
    dstp_slab = dstp.reshape(NW, CHUNKS, CHUNK)

    x_pad = jnp.zeros((N_PAD, D), jnp.float32).at[:N].set(x)
    b1r = b1.reshape(1, D)
    b2r = b2.reshape(1, D)
    b3r = b3.reshape(1, D)

    degp = _deg_kernel(dstp_slab)
    ds, di = _deg_combine(degp)
    ds_col = ds.reshape(N_PAD, 1)
    di_col = di.reshape(N_PAD, 1)

    h1, g1 = _mm1(x_pad, W1, ds_col)
    agg = _agg_kernel(g1, srcp, dstp).reshape(NC, N_PAD, D)
    h2, g2 = _comb(agg[0], agg[1], h1, ds_col, di_col, b1r, W2)
    agg = _agg_kernel(g2, srcp, dstp).reshape(NC, N_PAD, D)
    h3, g3 = _comb(agg[0], agg[1], h2, ds_col, di_col, b2r, W3)
    agg = _agg_kernel(g3, srcp, dstp).reshape(NC, N_PAD, D)
    out = _final(agg[0], agg[1], h3, ds_col, di_col, b3r)
    return out[:N]


# speedup vs baseline: 1.0335x; 1.0335x over previous
import functools

import jax
import jax.numpy as jnp
from jax import lax
from jax.experimental import pallas as pl
from jax.experimental.pallas import tpu as pltpu
from jax.experimental.pallas import tpu_sc as plsc

N = 10000
D = 128
E = 320000
NC = 2
NS = 16
NW = NC * NS
L = 16

N_PAD = 10240
ROWS_PER_TILE = N_PAD // NS
CHUNK = 128
E_TILE = 10240
CHUNKS = E_TILE // CHUNK
E_PAD = NW * E_TILE
TOTAL_CHUNKS = E_PAD // CHUNK
PAD_IDX = N
NBUF = 2
C0_CHUNKS = 142
C1_CHUNKS = 18

_mesh = plsc.VectorSubcoreMesh(core_axis_name="c", subcore_axis_name="s")
_sc_params = pltpu.CompilerParams(needs_layout_passes=False)


@functools.partial(
    pl.kernel,
    out_type=jax.ShapeDtypeStruct((NW, N_PAD), jnp.float32),
    mesh=_mesh,
    scratch_types=[
        pltpu.VMEM((CHUNKS, CHUNK), jnp.int32),
        pltpu.VMEM((N_PAD,), jnp.float32),
    ],
    compiler_params=_sc_params,
)
def _deg_kernel(dst_hbm, out_hbm, dst_v, deg_v):
    c = lax.axis_index("c")
    s = lax.axis_index("s")
    wid = s * NC + c

    zeros16 = jnp.zeros((L,), jnp.float32)

    def zero_body(i, _):
        deg_v[pl.ds(i * L, L)] = zeros16
        return 0

    lax.fori_loop(0, N_PAD // L, zero_body, 0)

    pltpu.sync_copy(dst_hbm.at[wid], dst_v)

    ones16 = jnp.ones((L,), jnp.float32)

    def chunk_body(j, _):
        for k in range(CHUNK // L):
            idx = dst_v[j, pl.ds(k * L, L)]
            plsc.addupdate_scatter(deg_v, [idx], ones16)
        return 0

    lax.fori_loop(0, CHUNKS, chunk_body, 0)

    pltpu.sync_copy(deg_v, out_hbm.at[wid])


@functools.partial(
    pl.kernel,
    out_type=jax.ShapeDtypeStruct((NW, ROWS_PER_TILE, D), jnp.float32),
    mesh=_mesh,
    scratch_types=[
        [pltpu.VMEM((CHUNK,), jnp.int32)] * NBUF,
        [pltpu.VMEM((CHUNK,), jnp.int32)] * NBUF,
        [pltpu.VMEM((CHUNK, D), jnp.float32)] * NBUF,
        pltpu.VMEM_SHARED((N_PAD, D), jnp.float32),
        [pltpu.SemaphoreType.DMA] * NBUF,
        [pltpu.SemaphoreType.DMA] * NBUF,
        [pltpu.SemaphoreType.DMA] * NBUF,
        [pltpu.SemaphoreType.DMA] * NBUF,
    ],
    compiler_params=_sc_params,
)
def _agg_kernel(g_hbm, src_hbm, dst_hbm, out_hbm, sidx, didx, bufs,
                acc, isems, dsems, gsems, ssems):
    c = lax.axis_index("c")
    s = lax.axis_index("s")
    owid = c * NS + s
    n = jnp.where(c == 0, C0_CHUNKS, C1_CHUNKS)
    base = jnp.where(c == 0, s * C0_CHUNKS,
                     NS * C0_CHUNKS + s * C1_CHUNKS)

    zeros16 = jnp.zeros((L,), jnp.float32)

    def zbuf_body(i, _):
        for k in range(D // L):
            bufs[0][i, pl.ds(k * L, L)] = zeros16
        return 0

    lax.fori_loop(0, CHUNK, zbuf_body, 0)

    def zacc_body(t, _):
        pltpu.sync_copy(
            bufs[0], acc.at[pl.ds(s * ROWS_PER_TILE + t * CHUNK, CHUNK)])
        return 0

    lax.fori_loop(0, ROWS_PER_TILE // CHUNK, zacc_body, 0)

    plsc.subcore_barrier()

    for b in range(NBUF):
        pltpu.async_copy(src_hbm.at[base + b], sidx[b], isems[b])
        pltpu.async_copy(dst_hbm.at[base + b], didx[b], dsems[b])
    for b in range(NBUF):
        pltpu.make_async_copy(src_hbm.at[base + b], sidx[b],
                              isems[b]).wait()
        pltpu.async_copy(g_hbm.at[sidx[b]], bufs[b], gsems[b])

    def step(jj, _):
        for b in range(NBUF):
            k = jj * NBUF + b
            j = base + k
            pltpu.make_async_copy(g_hbm.at[sidx[b]], bufs[b],
                                  gsems[b]).wait()
            pltpu.make_async_copy(dst_hbm.at[j], didx[b], dsems[b]).wait()
            pltpu.async_copy(bufs[b], acc.at[didx[b]], ssems[b], add=True)
            nk = k + NBUF

            @pl.when(nk < n)
            def _():
                pltpu.async_copy(src_hbm.at[j + NBUF], sidx[b], isems[b])
                pltpu.make_async_copy(bufs[b], acc.at[didx[b]],
                                      ssems[b]).wait()
                pltpu.async_copy(dst_hbm.at[j + NBUF], didx[b], dsems[b])
                pltpu.make_async_copy(src_hbm.at[j + NBUF], sidx[b],
                                      isems[b]).wait()
                pltpu.async_copy(g_hbm.at[sidx[b]], bufs[b], gsems[b])

        return 0

    lax.fori_loop(0, n // NBUF, step, 0)

    for b in range(NBUF):
        pltpu.make_async_copy(bufs[b], acc.at[didx[b]], ssems[b]).wait()

    plsc.subcore_barrier()

    pltpu.sync_copy(acc.at[pl.ds(s * ROWS_PER_TILE, ROWS_PER_TILE)],
                    out_hbm.at[owid])


def _deg_combine_body(degp_ref, ds_ref, di_ref):
    deg = jnp.sum(degp_ref[...], axis=0, keepdims=True) + 1.0
    ds_ref[...] = lax.rsqrt(deg)
    di_ref[...] = 1.0 / deg


_deg_combine = pl.pallas_call(
    _deg_combine_body,
    out_shape=[jax.ShapeDtypeStruct((1, N_PAD), jnp.float32)] * 2,
)

BLK = 512
GRID = N_PAD // BLK

_row_spec = pl.BlockSpec((BLK, D), lambda i: (i, 0))
_col_spec = pl.BlockSpec((BLK, 1), lambda i: (i, 0))
_w_spec = pl.BlockSpec((D, D), lambda i: (0, 0))
_b_spec = pl.BlockSpec((1, D), lambda i: (0, 0))


def _mm1_body(x_ref, w_ref, ds_ref, h_ref, g_ref):
    h = jnp.dot(x_ref[...], w_ref[...], preferred_element_type=jnp.float32,
                precision=lax.Precision.HIGHEST)
    h_ref[...] = h
    g_ref[...] = h * ds_ref[...]


_mm1 = pl.pallas_call(
    _mm1_body,
    grid=(GRID,),
    in_specs=[_row_spec, _w_spec, _col_spec],
    out_specs=[_row_spec, _row_spec],
    out_shape=[jax.ShapeDtypeStruct((N_PAD, D), jnp.float32)] * 2,
)


def _comb_body(a0_ref, a1_ref, h_ref, ds_ref, di_ref, b_ref, w_ref,
               hn_ref, gn_ref):
    x = (a0_ref[...] + a1_ref[...]) * ds_ref[...] \
        + h_ref[...] * di_ref[...] + b_ref[...]
    x = jnp.maximum(x, 0.0)
    hn = jnp.dot(x, w_ref[...], preferred_element_type=jnp.float32,
                 precision=lax.Precision.HIGHEST)
    hn_ref[...] = hn
    gn_ref[...] = hn * ds_ref[...]


_comb = pl.pallas_call(
    _comb_body,
    grid=(GRID,),
    in_specs=[_row_spec, _row_spec, _row_spec, _col_spec, _col_spec,
              _b_spec, _w_spec],
    out_specs=[_row_spec, _row_spec],
    out_shape=[jax.ShapeDtypeStruct((N_PAD, D), jnp.float32)] * 2,
)


def _final_body(a0_ref, a1_ref, h_ref, ds_ref, di_ref, b_ref, o_ref):
    o_ref[...] = (a0_ref[...] + a1_ref[...]) * ds_ref[...] \
        + h_ref[...] * di_ref[...] + b_ref[...]


_final = pl.pallas_call(
    _final_body,
    grid=(GRID,),
    in_specs=[_row_spec, _row_spec, _row_spec, _col_spec, _col_spec, _b_spec],
    out_specs=_row_spec,
    out_shape=jax.ShapeDtypeStruct((N_PAD, D), jnp.float32),
)


def kernel(x, edge_index, W1, b1, W2, b2, W3, b3):
    src = edge_index[0].astype(jnp.int32)
    dst = edge_index[1].astype(jnp.int32)
    pad = jnp.full((E_PAD - E,), PAD_IDX, jnp.int32)
    srcp = jnp.concatenate([src, pad]).reshape(TOTAL_CHUNKS, CHUNK)
    dstp = jnp.concatenate([dst, pad]).reshape(TOTAL_CHUNKS, CHUNK)
    dstp_slab = dstp.reshape(NW, CHUNKS, CHUNK)

    x_pad = jnp.zeros((N_PAD, D), jnp.float32).at[:N].set(x)
    b1r = b1.reshape(1, D)
    b2r = b2.reshape(1, D)
    b3r = b3.reshape(1, D)

    degp = _deg_kernel(dstp_slab)
    ds, di = _deg_combine(degp)
    ds_col = ds.reshape(N_PAD, 1)
    di_col = di.reshape(N_PAD, 1)

    h1, g1 = _mm1(x_pad, W1, ds_col)
    agg = _agg_kernel(g1, srcp, dstp).reshape(NC, N_PAD, D)
    h2, g2 = _comb(agg[0], agg[1], h1, ds_col, di_col, b1r, W2)
    agg = _agg_kernel(g2, srcp, dstp).reshape(NC, N_PAD, D)
    h3, g3 = _comb(agg[0], agg[1], h2, ds_col, di_col, b2r, W3)
    agg = _agg_kernel(g3, srcp, dstp).reshape(NC, N_PAD, D)
    out = _final(agg[0], agg[1], h3, ds_col, di_col, b3r)
    return out[:N]


# speedup vs baseline: 1.0438x; 1.0100x over previous
import functools

import jax
import jax.numpy as jnp
from jax import lax
from jax.experimental import pallas as pl
from jax.experimental.pallas import tpu as pltpu
from jax.experimental.pallas import tpu_sc as plsc

N = 10000
D = 128
E = 320000
NC = 2
NS = 16
NW = NC * NS
L = 16

N_PAD = 10240
ROWS_PER_TILE = N_PAD // NS
CHUNK = 128
E_TILE = 10240
CHUNKS = E_TILE // CHUNK
E_PAD = NW * E_TILE
TOTAL_CHUNKS = E_PAD // CHUNK
PAD_IDX = N
NBUF = 2
C0_CHUNKS = 152
C1_CHUNKS = 8

_mesh = plsc.VectorSubcoreMesh(core_axis_name="c", subcore_axis_name="s")
_sc_params = pltpu.CompilerParams(needs_layout_passes=False)


@functools.partial(
    pl.kernel,
    out_type=jax.ShapeDtypeStruct((NW, N_PAD), jnp.float32),
    mesh=_mesh,
    scratch_types=[
        pltpu.VMEM((CHUNKS, CHUNK), jnp.int32),
        pltpu.VMEM((N_PAD,), jnp.float32),
    ],
    compiler_params=_sc_params,
)
def _deg_kernel(dst_hbm, out_hbm, dst_v, deg_v):
    c = lax.axis_index("c")
    s = lax.axis_index("s")
    wid = s * NC + c

    zeros16 = jnp.zeros((L,), jnp.float32)

    def zero_body(i, _):
        deg_v[pl.ds(i * L, L)] = zeros16
        return 0

    lax.fori_loop(0, N_PAD // L, zero_body, 0)

    pltpu.sync_copy(dst_hbm.at[wid], dst_v)

    ones16 = jnp.ones((L,), jnp.float32)

    def chunk_body(j, _):
        for k in range(CHUNK // L):
            idx = dst_v[j, pl.ds(k * L, L)]
            plsc.addupdate_scatter(deg_v, [idx], ones16)
        return 0

    lax.fori_loop(0, CHUNKS, chunk_body, 0)

    pltpu.sync_copy(deg_v, out_hbm.at[wid])


@functools.partial(
    pl.kernel,
    out_type=jax.ShapeDtypeStruct((NW, ROWS_PER_TILE, D), jnp.float32),
    mesh=_mesh,
    scratch_types=[
        [pltpu.VMEM((CHUNK,), jnp.int32)] * NBUF,
        [pltpu.VMEM((CHUNK,), jnp.int32)] * NBUF,
        [pltpu.VMEM((CHUNK, D), jnp.float32)] * NBUF,
        pltpu.VMEM_SHARED((N_PAD, D), jnp.float32),
        [pltpu.SemaphoreType.DMA] * NBUF,
        [pltpu.SemaphoreType.DMA] * NBUF,
        [pltpu.SemaphoreType.DMA] * NBUF,
        [pltpu.SemaphoreType.DMA] * NBUF,
    ],
    compiler_params=_sc_params,
)
def _agg_kernel(g_hbm, src_hbm, dst_hbm, out_hbm, sidx, didx, bufs,
                acc, isems, dsems, gsems, ssems):
    c = lax.axis_index("c")
    s = lax.axis_index("s")
    owid = c * NS + s
    n = jnp.where(c == 0, C0_CHUNKS, C1_CHUNKS)
    base = jnp.where(c == 0, s * C0_CHUNKS,
                     NS * C0_CHUNKS + s * C1_CHUNKS)

    zeros16 = jnp.zeros((L,), jnp.float32)

    def zbuf_body(i, _):
        for k in range(D // L):
            bufs[0][i, pl.ds(k * L, L)] = zeros16
        return 0

    lax.fori_loop(0, CHUNK, zbuf_body, 0)

    def zacc_body(t, _):
        pltpu.sync_copy(
            bufs[0], acc.at[pl.ds(s * ROWS_PER_TILE + t * CHUNK, CHUNK)])
        return 0

    lax.fori_loop(0, ROWS_PER_TILE // CHUNK, zacc_body, 0)

    plsc.subcore_barrier()

    for b in range(NBUF):
        pltpu.async_copy(src_hbm.at[base + b], sidx[b], isems[b])
        pltpu.async_copy(dst_hbm.at[base + b], didx[b], dsems[b])
    for b in range(NBUF):
        pltpu.make_async_copy(src_hbm.at[base + b], sidx[b],
                              isems[b]).wait()
        pltpu.async_copy(g_hbm.at[sidx[b]], bufs[b], gsems[b])

    def step(jj, _):
        for b in range(NBUF):
            k = jj * NBUF + b
            j = base + k
            pltpu.make_async_copy(g_hbm.at[sidx[b]], bufs[b],
                                  gsems[b]).wait()
            pltpu.make_async_copy(dst_hbm.at[j], didx[b], dsems[b]).wait()
            pltpu.async_copy(bufs[b], acc.at[didx[b]], ssems[b], add=True)
            nk = k + NBUF

            @pl.when(nk < n)
            def _():
                pltpu.async_copy(src_hbm.at[j + NBUF], sidx[b], isems[b])
                pltpu.make_async_copy(bufs[b], acc.at[didx[b]],
                                      ssems[b]).wait()
                pltpu.async_copy(dst_hbm.at[j + NBUF], didx[b], dsems[b])
                pltpu.make_async_copy(src_hbm.at[j + NBUF], sidx[b],
                                      isems[b]).wait()
                pltpu.async_copy(g_hbm.at[sidx[b]], bufs[b], gsems[b])

        return 0

    lax.fori_loop(0, n // NBUF, step, 0)

    for b in range(NBUF):
        pltpu.make_async_copy(bufs[b], acc.at[didx[b]], ssems[b]).wait()

    plsc.subcore_barrier()

    pltpu.sync_copy(acc.at[pl.ds(s * ROWS_PER_TILE, ROWS_PER_TILE)],
                    out_hbm.at[owid])


def _deg_combine_body(degp_ref, ds_ref, di_ref):
    deg = jnp.sum(degp_ref[...], axis=0, keepdims=True) + 1.0
    ds_ref[...] = lax.rsqrt(deg)
    di_ref[...] = 1.0 / deg


_deg_combine = pl.pallas_call(
    _deg_combine_body,
    out_shape=[jax.ShapeDtypeStruct((1, N_PAD), jnp.float32)] * 2,
)

BLK = 512
GRID = N_PAD // BLK

_row_spec = pl.BlockSpec((BLK, D), lambda i: (i, 0))
_col_spec = pl.BlockSpec((BLK, 1), lambda i: (i, 0))
_w_spec = pl.BlockSpec((D, D), lambda i: (0, 0))
_b_spec = pl.BlockSpec((1, D), lambda i: (0, 0))


def _mm1_body(x_ref, w_ref, ds_ref, h_ref, g_ref):
    h = jnp.dot(x_ref[...], w_ref[...], preferred_element_type=jnp.float32,
                precision=lax.Precision.HIGHEST)
    h_ref[...] = h
    g_ref[...] = h * ds_ref[...]


_mm1 = pl.pallas_call(
    _mm1_body,
    grid=(GRID,),
    in_specs=[_row_spec, _w_spec, _col_spec],
    out_specs=[_row_spec, _row_spec],
    out_shape=[jax.ShapeDtypeStruct((N_PAD, D), jnp.float32)] * 2,
)


def _comb_body(a0_ref, a1_ref, h_ref, ds_ref, di_ref, b_ref, w_ref,
               hn_ref, gn_ref):
    x = (a0_ref[...] + a1_ref[...]) * ds_ref[...] \
        + h_ref[...] * di_ref[...] + b_ref[...]
    x = jnp.maximum(x, 0.0)
    hn = jnp.dot(x, w_ref[...], preferred_element_type=jnp.float32,
                 precision=lax.Precision.HIGHEST)
    hn_ref[...] = hn
    gn_ref[...] = hn * ds_ref[...]


_comb = pl.pallas_call(
    _comb_body,
    grid=(GRID,),
    in_specs=[_row_spec, _row_spec, _row_spec, _col_spec, _col_spec,
              _b_spec, _w_spec],
    out_specs=[_row_spec, _row_spec],
    out_shape=[jax.ShapeDtypeStruct((N_PAD, D), jnp.float32)] * 2,
)


def _final_body(a0_ref, a1_ref, h_ref, ds_ref, di_ref, b_ref, o_ref):
    o_ref[...] = (a0_ref[...] + a1_ref[...]) * ds_ref[...] \
        + h_ref[...] * di_ref[...] + b_ref[...]


_final = pl.pallas_call(
    _final_body,
    grid=(GRID,),
    in_specs=[_row_spec, _row_spec, _row_spec, _col_spec, _col_spec, _b_spec],
    out_specs=_row_spec,
    out_shape=jax.ShapeDtypeStruct((N_PAD, D), jnp.float32),
)


def kernel(x, edge_index, W1, b1, W2, b2, W3, b3):
    src = edge_index[0].astype(jnp.int32)
    dst = edge_index[1].astype(jnp.int32)
    pad = jnp.full((E_PAD - E,), PAD_IDX, jnp.int32)
    srcp = jnp.concatenate([src, pad]).reshape(TOTAL_CHUNKS, CHUNK)
    dstp = jnp.concatenate([dst, pad]).reshape(TOTAL_CHUNKS, CHUNK)
    dstp_slab = dstp.reshape(NW, CHUNKS, CHUNK)

    x_pad = jnp.zeros((N_PAD, D), jnp.float32).at[:N].set(x)
    b1r = b1.reshape(1, D)
    b2r = b2.reshape(1, D)
    b3r = b3.reshape(1, D)

    degp = _deg_kernel(dstp_slab)
    ds, di = _deg_combine(degp)
    ds_col = ds.reshape(N_PAD, 1)
    di_col = di.reshape(N_PAD, 1)

    h1, g1 = _mm1(x_pad, W1, ds_col)
    agg = _agg_kernel(g1, srcp, dstp).reshape(NC, N_PAD, D)
    h2, g2 = _comb(agg[0], agg[1], h1, ds_col, di_col, b1r, W2)
    agg = _agg_kernel(g2, srcp, dstp).reshape(NC, N_PAD, D)
    h3, g3 = _comb(agg[0], agg[1], h2, ds_col, di_col, b2r, W3)
    agg = _agg_kernel(g3, srcp, dstp).reshape(NC, N_PAD, D)
    out = _final(agg[0], agg[1], h3, ds_col, di_col, b3r)
    return out[:N]
